# baseline (speedup 1.0000x reference)
import functools

import jax
import jax.numpy as jnp
from jax import lax
from jax.experimental import pallas as pl
from jax.experimental.pallas import tpu as pltpu
from jax.experimental.pallas import tpu_sc as plsc

N_ROWS = 160000
N_SEG = 10000
D = 256
DH = 128
CHUNK = 64
N_CHUNKS = N_ROWS // CHUNK
NS = 16
NBUF = 6
CHUNKS_PER_TILE = -(-N_CHUNKS // NS)
SEG_PER_TILE = 624
SEG_TAIL = N_SEG - NS * SEG_PER_TILE

_mesh = plsc.VectorSubcoreMesh(core_axis_name="c", subcore_axis_name="s")


def _body(feat_hbm, idx_hbm, out_hbm, *rest):
    idx_bufs = rest[:NBUF]
    row_bufs = rest[NBUF:2 * NBUF]
    acc_sh = rest[2 * NBUF]
    lsems = rest[2 * NBUF + 1:2 * NBUF + 1 + NBUF]
    ssems = rest[3 * NBUF + 1:4 * NBUF + 1]
    zsem = rest[4 * NBUF + 1]
    rows0 = row_bufs[0]

    c = lax.axis_index("c")
    s = lax.axis_index("s")
    col0 = c * DH

    zeros16 = jnp.zeros((16,), jnp.float32)

    def zrow(r, carry):
        for k in range(DH // 16):
            rows0[r, pl.ds(k * 16, 16)] = zeros16
        return carry

    lax.fori_loop(0, CHUNK, zrow, 0)

    seg_base = s * SEG_PER_TILE
    full = SEG_PER_TILE // CHUNK
    rem = SEG_PER_TILE - full * CHUNK
    for j in range(full):
        pltpu.async_copy(rows0, acc_sh.at[pl.ds(seg_base + j * CHUNK, CHUNK)],
                         zsem)
    pltpu.async_copy(rows0.at[pl.ds(0, rem)],
                     acc_sh.at[pl.ds(seg_base + full * CHUNK, rem)], zsem)

    @pl.when(s == 0)
    def _():
        pltpu.sync_copy(rows0.at[pl.ds(0, SEG_TAIL)],
                        acc_sh.at[pl.ds(NS * SEG_PER_TILE, SEG_TAIL)])

    for b in range(1, NBUF):
        ch0 = s + b * NS
        pltpu.async_copy(idx_hbm.at[pl.ds(ch0 * CHUNK, CHUNK)],
                         idx_bufs[b], lsems[b])
        pltpu.async_copy(
            feat_hbm.at[pl.ds(ch0 * CHUNK, CHUNK), pl.ds(col0, DH)],
            row_bufs[b], lsems[b])

    for j in range(full):
        pltpu.make_async_copy(
            rows0, acc_sh.at[pl.ds(seg_base + j * CHUNK, CHUNK)],
            zsem).wait()
    pltpu.make_async_copy(
        rows0.at[pl.ds(0, rem)],
        acc_sh.at[pl.ds(seg_base + full * CHUNK, rem)], zsem).wait()
    plsc.subcore_barrier()

    def load_issue(i, b):
        ch = s + i * NS

        @pl.when(ch < N_CHUNKS)
        def _():
            rbase = ch * CHUNK

            @pl.when(i >= NBUF)
            def _():
                pltpu.make_async_copy(
                    row_bufs[b], acc_sh.at[idx_bufs[b]], ssems[b]).wait()

            pltpu.async_copy(idx_hbm.at[pl.ds(rbase, CHUNK)],
                             idx_bufs[b], lsems[b])
            pltpu.async_copy(
                feat_hbm.at[pl.ds(rbase, CHUNK), pl.ds(col0, DH)],
                row_bufs[b], lsems[b])

    def scatter_issue(j, bj):
        chj = s + j * NS

        @pl.when(jnp.logical_and(j >= 0, chj < N_CHUNKS))
        def _():
            rbase = chj * CHUNK
            pltpu.make_async_copy(idx_hbm.at[pl.ds(rbase, CHUNK)],
                                  idx_bufs[bj], lsems[bj]).wait()
            pltpu.make_async_copy(
                feat_hbm.at[pl.ds(rbase, CHUNK), pl.ds(col0, DH)],
                row_bufs[bj], lsems[bj]).wait()
            pltpu.async_copy(row_bufs[bj], acc_sh.at[idx_bufs[bj]],
                             ssems[bj], add=True)

    n_steps = CHUNKS_PER_TILE + NBUF - 1
    n_super = -(-n_steps // NBUF)

    def super_body(t, carry):
        for b in range(NBUF):
            i = t * NBUF + b
            load_issue(i, b)
            scatter_issue(i - (NBUF - 1), (b + 1) % NBUF)
        return carry

    load_issue(0, 0)
    scatter_issue(0, 0)
    lax.fori_loop(1, n_super, super_body, 0)

    for b in range(NBUF):
        pltpu.make_async_copy(row_bufs[b], acc_sh.at[idx_bufs[b]],
                              ssems[b]).wait()

    plsc.subcore_barrier()

    pltpu.sync_copy(acc_sh.at[pl.ds(seg_base, SEG_PER_TILE)],
                    out_hbm.at[pl.ds(seg_base, SEG_PER_TILE), pl.ds(col0, DH)])

    @pl.when(s == 0)
    def _():
        pltpu.sync_copy(
            acc_sh.at[pl.ds(NS * SEG_PER_TILE, SEG_TAIL)],
            out_hbm.at[pl.ds(NS * SEG_PER_TILE, SEG_TAIL), pl.ds(col0, DH)])


_seg_sum = functools.partial(
    pl.kernel,
    mesh=_mesh,
    out_type=jax.ShapeDtypeStruct((N_SEG, D), jnp.float32),
    scratch_types=(
        [pltpu.VMEM((CHUNK,), jnp.int32) for _ in range(NBUF)]
        + [pltpu.VMEM((CHUNK, DH), jnp.float32) for _ in range(NBUF)]
        + [pltpu.VMEM_SHARED((N_SEG, DH), jnp.float32)]
        + [pltpu.SemaphoreType.DMA for _ in range(2 * NBUF + 1)]
    ),
)(_body)


@jax.jit
def kernel(features, structural_indices):
    idx = structural_indices.astype(jnp.int32)
    return _seg_sum(features, idx)


# speedup vs baseline: 1.0137x; 1.0137x over previous
import functools

import jax
import jax.numpy as jnp
from jax import lax
from jax.experimental import pallas as pl
from jax.experimental.pallas import tpu as pltpu
from jax.experimental.pallas import tpu_sc as plsc

N_ROWS = 160000
N_SEG = 10000
D = 256
DH = 128
CHUNK = 128
N_CHUNKS = N_ROWS // CHUNK
NS = 16
NBUF = 3
CHUNKS_PER_TILE = -(-N_CHUNKS // NS)
SEG_PER_TILE = 624
SEG_TAIL = N_SEG - NS * SEG_PER_TILE

_mesh = plsc.VectorSubcoreMesh(core_axis_name="c", subcore_axis_name="s")


def _body(feat_hbm, idx_hbm, out_hbm, *rest):
    idx_bufs = rest[:NBUF]
    row_bufs = rest[NBUF:2 * NBUF]
    acc_sh = rest[2 * NBUF]
    lsems = rest[2 * NBUF + 1:2 * NBUF + 1 + NBUF]
    ssems = rest[2 * NBUF + 1 + NBUF:]
    rows0 = row_bufs[0]

    c = lax.axis_index("c")
    s = lax.axis_index("s")
    col0 = c * DH

    zeros16 = jnp.zeros((16,), jnp.float32)

    def zrow(r, carry):
        for k in range(DH // 16):
            rows0[r, pl.ds(k * 16, 16)] = zeros16
        return carry

    lax.fori_loop(0, CHUNK, zrow, 0)

    seg_base = s * SEG_PER_TILE
    full = SEG_PER_TILE // CHUNK
    rem = SEG_PER_TILE - full * CHUNK
    for j in range(full):
        pltpu.sync_copy(rows0, acc_sh.at[pl.ds(seg_base + j * CHUNK, CHUNK)])
    pltpu.sync_copy(rows0.at[pl.ds(0, rem)],
                    acc_sh.at[pl.ds(seg_base + full * CHUNK, rem)])

    @pl.when(s == 0)
    def _():
        pltpu.sync_copy(rows0.at[pl.ds(0, SEG_TAIL)],
                        acc_sh.at[pl.ds(NS * SEG_PER_TILE, SEG_TAIL)])

    plsc.subcore_barrier()

    def load_issue(i, b):
        ch = s + i * NS

        @pl.when(ch < N_CHUNKS)
        def _():
            rbase = ch * CHUNK

            @pl.when(i >= NBUF)
            def _():
                pltpu.make_async_copy(
                    row_bufs[b], acc_sh.at[idx_bufs[b]], ssems[b]).wait()

            pltpu.async_copy(idx_hbm.at[pl.ds(rbase, CHUNK)],
                             idx_bufs[b], lsems[b])
            pltpu.async_copy(
                feat_hbm.at[pl.ds(rbase, CHUNK), pl.ds(col0, DH)],
                row_bufs[b], lsems[b])

    def scatter_issue(j, bj):
        chj = s + j * NS

        @pl.when(jnp.logical_and(j >= 0, chj < N_CHUNKS))
        def _():
            rbase = chj * CHUNK
            pltpu.make_async_copy(idx_hbm.at[pl.ds(rbase, CHUNK)],
                                  idx_bufs[bj], lsems[bj]).wait()
            pltpu.make_async_copy(
                feat_hbm.at[pl.ds(rbase, CHUNK), pl.ds(col0, DH)],
                row_bufs[bj], lsems[bj]).wait()
            pltpu.async_copy(row_bufs[bj], acc_sh.at[idx_bufs[bj]],
                             ssems[bj], add=True)

    n_steps = CHUNKS_PER_TILE + NBUF - 1
    n_super = -(-n_steps // NBUF)

    def super_body(t, carry):
        for b in range(NBUF):
            i = t * NBUF + b
            load_issue(i, b)
            scatter_issue(i - (NBUF - 1), (b + 1) % NBUF)
        return carry

    lax.fori_loop(0, n_super, super_body, 0)

    for b in range(NBUF):
        pltpu.make_async_copy(row_bufs[b], acc_sh.at[idx_bufs[b]],
                              ssems[b]).wait()

    plsc.subcore_barrier()

    pltpu.sync_copy(acc_sh.at[pl.ds(seg_base, SEG_PER_TILE)],
                    out_hbm.at[pl.ds(seg_base, SEG_PER_TILE), pl.ds(col0, DH)])

    @pl.when(s == 0)
    def _():
        pltpu.sync_copy(
            acc_sh.at[pl.ds(NS * SEG_PER_TILE, SEG_TAIL)],
            out_hbm.at[pl.ds(NS * SEG_PER_TILE, SEG_TAIL), pl.ds(col0, DH)])


_seg_sum = functools.partial(
    pl.kernel,
    mesh=_mesh,
    out_type=jax.ShapeDtypeStruct((N_SEG, D), jnp.float32),
    scratch_types=(
        [pltpu.VMEM((CHUNK,), jnp.int32) for _ in range(NBUF)]
        + [pltpu.VMEM((CHUNK, DH), jnp.float32) for _ in range(NBUF)]
        + [pltpu.VMEM_SHARED((N_SEG, DH), jnp.float32)]
        + [pltpu.SemaphoreType.DMA for _ in range(2 * NBUF)]
    ),
)(_body)


@jax.jit
def kernel(features, structural_indices):
    idx = structural_indices.astype(jnp.int32)
    return _seg_sum(features, idx)
